# baseline (speedup 1.0000x reference)
import functools

import jax
import jax.numpy as jnp
from jax import lax
from jax.experimental import pallas as pl
from jax.experimental.pallas import tpu as pltpu
from jax.experimental.pallas import tpu_sc as plsc

NB = 16384
NF = 26
D = 32
NC = 2
NS = 16
NW = NC * NS
SLAB = NB // NW
NBUF = 2

_mesh = plsc.VectorSubcoreMesh(core_axis_name="c", subcore_axis_name="s")


@functools.partial(
    pl.kernel,
    mesh=_mesh,
    out_type=jax.ShapeDtypeStruct((NF, D, NB), jnp.float32),
    scratch_types=[
        [pltpu.VMEM((SLAB,), jnp.int32) for _ in range(NBUF)],
        [pltpu.VMEM((SLAB, D), jnp.float32) for _ in range(NBUF)],
        [pltpu.VMEM((D, SLAB), jnp.float32) for _ in range(NBUF)],
        [pltpu.SemaphoreType.DMA for _ in range(NBUF)],
        [pltpu.SemaphoreType.DMA for _ in range(NBUF)],
        [pltpu.SemaphoreType.DMA for _ in range(NBUF)],
    ],
    compiler_params=pltpu.CompilerParams(use_tc_tiling_on_sc=False,
                                         needs_layout_passes=False),
)
def _gather_kernel(ix_hbm, w_hbm, out_hbm, idxs, rows, trs, isems, gsems, wsems):
    wid = lax.axis_index("s") * NC + lax.axis_index("c")
    b0 = pl.multiple_of(wid * SLAB, 8)

    def fire_unit(f, b):
        pltpu.sync_copy(ix_hbm.at[pl.ds(f * NB + b0, SLAB)], idxs[b])
        pltpu.async_copy(w_hbm.at[idxs[b]], rows[b], gsems[b])

    def wait_gather(b):
        pltpu.make_async_copy(w_hbm.at[idxs[b]], rows[b], gsems[b]).wait()

    def transpose(b):
        lane = lax.iota(jnp.int32, 16)

        def e_body(e, carry):
            col = jnp.full((16,), e, dtype=jnp.int32)
            for bg in range(SLAB // 16):
                row = bg * 16 + lane
                v = plsc.load_gather(rows[b], [row, col])
                trs[b][e, pl.ds(bg * 16, 16)] = v
            return carry

        lax.fori_loop(0, D, e_body, 0)

    def fire_write(f, b):
        pltpu.async_copy(trs[b], out_hbm.at[f, :, pl.ds(b0, SLAB)], wsems[b])

    def wait_write(f, b):
        pltpu.make_async_copy(trs[b], out_hbm.at[f, :, pl.ds(b0, SLAB)],
                              wsems[b]).wait()

    fire_unit(0, 0)
    for pair in range(NF // NBUF):
        f0, f1 = 2 * pair, 2 * pair + 1
        fire_unit(f1, 1)
        wait_gather(0)
        if pair > 0:
            wait_write(f0 - 2, 0)
        transpose(0)
        fire_write(f0, 0)
        if f1 + 1 < NF:
            fire_unit(f1 + 1, 0)
        wait_gather(1)
        if pair > 0:
            wait_write(f1 - 2, 1)
        transpose(1)
        fire_write(f1, 1)
    wait_write(NF - 2, 0)
    wait_write(NF - 1, 1)


def kernel(IX, weight):
    nb, nf = IX.shape
    flat = IX.T.reshape(-1).astype(jnp.int32)
    out = _gather_kernel(flat, weight)
    return out.transpose(2, 0, 1)
